# baseline (speedup 1.0000x reference)
import functools

import jax
import jax.numpy as jnp
from jax import lax
from jax.experimental import pallas as pl
from jax.experimental.pallas import tpu as pltpu
from jax.experimental.pallas import tpu_sc as plsc

N = 10000
D = 128
NC = 2
NS = 16
NW = NC * NS
CHUNK = 128
NPAD = 10240
RPT = NPAD // NS
BLK = 1024

_mesh = plsc.VectorSubcoreMesh(core_axis_name="c", subcore_axis_name="s",
                               num_cores=NC, num_subcores=NS)


def _fill(ref, n, value):
  v = jnp.full((16,), value, jnp.float32)

  def body(i, _):
    ref[pl.ds(i * 16, 16)] = v
    return 0

  lax.fori_loop(0, n // 16, body, 0)


def _deg_body(nch, row_hbm, dp_hbm, row_scr, ones_v, dbuf, deg_sh):
  c = lax.axis_index("c")
  s = lax.axis_index("s")
  w = c * NS + s
  _fill(dbuf, RPT, 0.0)
  _fill(ones_v, CHUNK, 1.0)
  pltpu.sync_copy(dbuf, deg_sh.at[pl.ds(s * RPT, RPT)])
  plsc.subcore_barrier()
  pltpu.sync_copy(row_hbm.at[w], row_scr)

  def body(j, _):
    pltpu.sync_copy(ones_v, deg_sh.at[row_scr.at[j]], add=True)
    return 0

  lax.fori_loop(0, nch, body, 0)
  plsc.subcore_barrier()
  pltpu.sync_copy(deg_sh.at[pl.ds(s * RPT, RPT)], dbuf)
  pltpu.sync_copy(dbuf, dp_hbm.at[c, pl.ds(s * RPT, RPT)])


BCH = 16


def _scat_body(nch, hs_hbm, row_hbm, col_hbm, p_hbm,
               row_scr, col_scr, b0, acc_sh, s0):
  c = lax.axis_index("c")
  s = lax.axis_index("s")
  w = c * NS + s
  def zbody(i, _):
    b0[i // 8, pl.ds((i % 8) * 16, 16)] = jnp.zeros((16,), jnp.float32)
    return 0

  lax.fori_loop(0, CHUNK * 8, zbody, 0)

  def ibody(k, _):
    pltpu.sync_copy(b0, acc_sh.at[pl.ds(s * RPT + k * CHUNK, CHUNK)])
    return 0

  lax.fori_loop(0, RPT // CHUNK, ibody, 0)
  plsc.subcore_barrier()

  pltpu.sync_copy(row_hbm.at[w], row_scr)
  pltpu.sync_copy(col_hbm.at[w], col_scr)

  def body(j, _):
    pltpu.async_copy(hs_hbm.at[col_scr.at[j]], b0, s0).wait()
    pltpu.sync_copy(b0, acc_sh.at[row_scr.at[j]], add=True)
    return 0

  lax.fori_loop(0, nch, body, 0)
  plsc.subcore_barrier()

  def ebody(k, _):
    pltpu.sync_copy(acc_sh.at[pl.ds(s * RPT + k * CHUNK, CHUNK)], b0)
    pltpu.sync_copy(b0, p_hbm.at[c, pl.ds(s * RPT + k * CHUNK, CHUNK)])
    return 0

  lax.fori_loop(0, RPT // CHUNK, ebody, 0)


def _mm_body(x_ref, w_ref, dp_ref, hs_ref):
  i = pl.program_id(0)
  h = jnp.dot(x_ref[...], w_ref[...], preferred_element_type=jnp.float32)
  deg = dp_ref[0, pl.ds(i * BLK, BLK)] + dp_ref[1, pl.ds(i * BLK, BLK)] + 1.0
  dis = lax.rsqrt(deg)
  hs_ref[...] = dis[:, None] * h


def _fin_body(hs_ref, p_ref, dp_ref, b_ref, o_ref):
  i = pl.program_id(0)
  deg = dp_ref[0, pl.ds(i * BLK, BLK)] + dp_ref[1, pl.ds(i * BLK, BLK)] + 1.0
  dis = lax.rsqrt(deg)
  s = hs_ref[...] + p_ref[0] + p_ref[1]
  o_ref[...] = dis[:, None] * s + b_ref[...][None, :]


def kernel(x, edge_index, W, b):
  e = edge_index.shape[1]
  nch = -(-e // (NW * CHUNK))
  nch = -(-nch // BCH) * BCH
  epad = NW * nch * CHUNK
  row = edge_index[0].astype(jnp.int32)
  col = edge_index[1].astype(jnp.int32)
  rowp = jnp.concatenate(
      [row, jnp.full((epad - e,), N, jnp.int32)]).reshape(NW, nch, CHUNK)
  colp = jnp.concatenate(
      [col, jnp.zeros((epad - e,), jnp.int32)]).reshape(NW, nch, CHUNK)

  deg_call = pl.kernel(
      functools.partial(_deg_body, nch),
      out_type=jax.ShapeDtypeStruct((NC, NPAD), jnp.float32),
      mesh=_mesh,
      scratch_types=[
          pltpu.VMEM((nch, CHUNK), jnp.int32),
          pltpu.VMEM((CHUNK,), jnp.float32),
          pltpu.VMEM((RPT,), jnp.float32),
          pltpu.VMEM_SHARED((NPAD,), jnp.float32),
      ],
  )
  dp = deg_call(rowp)

  xp = jnp.concatenate([x, jnp.zeros((NPAD - N, D), x.dtype)], axis=0)
  hs = pl.pallas_call(
      _mm_body,
      out_shape=jax.ShapeDtypeStruct((NPAD, D), jnp.float32),
      grid=(NPAD // BLK,),
      in_specs=[
          pl.BlockSpec((BLK, D), lambda i: (i, 0)),
          pl.BlockSpec((D, D), lambda i: (0, 0)),
          pl.BlockSpec((NC, NPAD), lambda i: (0, 0)),
      ],
      out_specs=pl.BlockSpec((BLK, D), lambda i: (i, 0)),
  )(xp, W, dp)

  scat_call = pl.kernel(
      functools.partial(_scat_body, nch),
      out_type=jax.ShapeDtypeStruct((NC, NPAD, D), jnp.float32),
      mesh=_mesh,
      scratch_types=[
          pltpu.VMEM((nch, CHUNK), jnp.int32),
          pltpu.VMEM((nch, CHUNK), jnp.int32),
          pltpu.VMEM((CHUNK, D), jnp.float32),
          pltpu.VMEM_SHARED((NPAD, D), jnp.float32),
          pltpu.SemaphoreType.DMA,
      ],
  )
  p = scat_call(hs, rowp, colp)

  out = pl.pallas_call(
      _fin_body,
      out_shape=jax.ShapeDtypeStruct((NPAD, D), jnp.float32),
      grid=(NPAD // BLK,),
      in_specs=[
          pl.BlockSpec((BLK, D), lambda i: (i, 0)),
          pl.BlockSpec((NC, BLK, D), lambda i: (0, i, 0)),
          pl.BlockSpec((NC, NPAD), lambda i: (0, 0)),
          pl.BlockSpec((D,), lambda i: (0,)),
      ],
      out_specs=pl.BlockSpec((BLK, D), lambda i: (i, 0)),
  )(hs, p, dp, b)
  return out[:N]


# speedup vs baseline: 2.3942x; 2.3942x over previous
import functools

import jax
import jax.numpy as jnp
from jax import lax
from jax.experimental import pallas as pl
from jax.experimental.pallas import tpu as pltpu
from jax.experimental.pallas import tpu_sc as plsc

N = 10000
D = 128
NC = 2
NS = 16
NW = NC * NS
CHUNK = 128
NPAD = 10240
RPT = NPAD // NS
BLK = 1024

_mesh = plsc.VectorSubcoreMesh(core_axis_name="c", subcore_axis_name="s",
                               num_cores=NC, num_subcores=NS)


def _fill(ref, n, value):
  v = jnp.full((16,), value, jnp.float32)

  def body(i, _):
    ref[pl.ds(i * 16, 16)] = v
    return 0

  lax.fori_loop(0, n // 16, body, 0)


def _deg_body(nch, row_hbm, dp_hbm, row_scr, ones_v, dbuf, deg_sh):
  c = lax.axis_index("c")
  s = lax.axis_index("s")
  w = c * NS + s
  _fill(dbuf, RPT, 0.0)
  _fill(ones_v, CHUNK, 1.0)
  pltpu.sync_copy(dbuf, deg_sh.at[pl.ds(s * RPT, RPT)])
  plsc.subcore_barrier()
  pltpu.sync_copy(row_hbm.at[w], row_scr)

  def body(j, _):
    pltpu.sync_copy(ones_v, deg_sh.at[row_scr.at[j]], add=True)
    return 0

  lax.fori_loop(0, nch, body, 0)
  plsc.subcore_barrier()
  pltpu.sync_copy(deg_sh.at[pl.ds(s * RPT, RPT)], dbuf)
  pltpu.sync_copy(dbuf, dp_hbm.at[c, pl.ds(s * RPT, RPT)])


BCH = 16


def _scat_body(nch, hs_hbm, row_hbm, col_hbm, p_hbm,
               row_scr, col_scr, b0, acc_sh, s0):
  c = lax.axis_index("c")
  s = lax.axis_index("s")
  w = c * NS + s
  def zbody(i, _):
    b0[i // 8, pl.ds((i % 8) * 16, 16)] = jnp.zeros((16,), jnp.float32)
    return 0

  lax.fori_loop(0, CHUNK * 8, zbody, 0)

  def ibody(k, _):
    pltpu.sync_copy(b0, acc_sh.at[pl.ds(s * RPT + k * CHUNK, CHUNK)])
    return 0

  lax.fori_loop(0, RPT // CHUNK, ibody, 0)
  plsc.subcore_barrier()

  pltpu.sync_copy(row_hbm.at[w], row_scr)
  pltpu.sync_copy(col_hbm.at[w], col_scr)

  def body(j, _):
    pltpu.async_copy(hs_hbm.at[col_scr.at[j]], b0, s0).wait()
    pltpu.sync_copy(b0, acc_sh.at[row_scr.at[j]], add=True)
    return 0

  lax.fori_loop(0, nch, body, 0)
  plsc.subcore_barrier()

  def ebody(k, _):
    pltpu.sync_copy(acc_sh.at[pl.ds(s * RPT + k * CHUNK, CHUNK)], b0)
    pltpu.sync_copy(b0, p_hbm.at[c, pl.ds(s * RPT + k * CHUNK, CHUNK)])
    return 0

  lax.fori_loop(0, RPT // CHUNK, ebody, 0)


def _mm_body(x_ref, w_ref, dp_ref, hs_ref):
  i = pl.program_id(0)
  h = jnp.dot(x_ref[...], w_ref[...], preferred_element_type=jnp.float32)
  deg = dp_ref[0, pl.ds(i * BLK, BLK)] + dp_ref[1, pl.ds(i * BLK, BLK)] + 1.0
  dis = lax.rsqrt(deg)
  hs_ref[...] = dis[:, None] * h


def _fin_body(hs_ref, p_ref, dp_ref, b_ref, o_ref):
  i = pl.program_id(0)
  deg = dp_ref[0, pl.ds(i * BLK, BLK)] + dp_ref[1, pl.ds(i * BLK, BLK)] + 1.0
  dis = lax.rsqrt(deg)
  s = hs_ref[...] + p_ref[0] + p_ref[1]
  o_ref[...] = dis[:, None] * s + b_ref[...][None, :]


def kernel(x, edge_index, W, b):
  e = edge_index.shape[1]
  nch = -(-e // (NW * CHUNK))
  nch = -(-nch // BCH) * BCH
  epad = NW * nch * CHUNK
  row = edge_index[0].astype(jnp.int32)
  col = edge_index[1].astype(jnp.int32)
  pad_ix = jnp.arange(epad - e, dtype=jnp.int32)
  rowp = jnp.concatenate(
      [row, N + pad_ix % (NPAD - N)]).reshape(NW, nch, CHUNK)
  colp = jnp.concatenate(
      [col, pad_ix % N]).reshape(NW, nch, CHUNK)

  deg_call = pl.kernel(
      functools.partial(_deg_body, nch),
      out_type=jax.ShapeDtypeStruct((NC, NPAD), jnp.float32),
      mesh=_mesh,
      scratch_types=[
          pltpu.VMEM((nch, CHUNK), jnp.int32),
          pltpu.VMEM((CHUNK,), jnp.float32),
          pltpu.VMEM((RPT,), jnp.float32),
          pltpu.VMEM_SHARED((NPAD,), jnp.float32),
      ],
  )
  dp = deg_call(rowp)

  xp = jnp.concatenate([x, jnp.zeros((NPAD - N, D), x.dtype)], axis=0)
  hs = pl.pallas_call(
      _mm_body,
      out_shape=jax.ShapeDtypeStruct((NPAD, D), jnp.float32),
      grid=(NPAD // BLK,),
      in_specs=[
          pl.BlockSpec((BLK, D), lambda i: (i, 0)),
          pl.BlockSpec((D, D), lambda i: (0, 0)),
          pl.BlockSpec((NC, NPAD), lambda i: (0, 0)),
      ],
      out_specs=pl.BlockSpec((BLK, D), lambda i: (i, 0)),
  )(xp, W, dp)

  scat_call = pl.kernel(
      functools.partial(_scat_body, nch),
      out_type=jax.ShapeDtypeStruct((NC, NPAD, D), jnp.float32),
      mesh=_mesh,
      scratch_types=[
          pltpu.VMEM((nch, CHUNK), jnp.int32),
          pltpu.VMEM((nch, CHUNK), jnp.int32),
          pltpu.VMEM((CHUNK, D), jnp.float32),
          pltpu.VMEM_SHARED((NPAD, D), jnp.float32),
          pltpu.SemaphoreType.DMA,
      ],
  )
  p = scat_call(hs, rowp, colp)

  out = pl.pallas_call(
      _fin_body,
      out_shape=jax.ShapeDtypeStruct((NPAD, D), jnp.float32),
      grid=(NPAD // BLK,),
      in_specs=[
          pl.BlockSpec((BLK, D), lambda i: (i, 0)),
          pl.BlockSpec((NC, BLK, D), lambda i: (0, i, 0)),
          pl.BlockSpec((NC, NPAD), lambda i: (0, 0)),
          pl.BlockSpec((D,), lambda i: (0,)),
      ],
      out_specs=pl.BlockSpec((BLK, D), lambda i: (i, 0)),
  )(hs, p, dp, b)
  return out[:N]


# speedup vs baseline: 3.1340x; 1.3090x over previous
import functools

import jax
import jax.numpy as jnp
from jax import lax
from jax.experimental import pallas as pl
from jax.experimental.pallas import tpu as pltpu
from jax.experimental.pallas import tpu_sc as plsc

N = 10000
D = 128
NC = 2
NS = 16
NW = NC * NS
CHUNK = 128
NPAD = 10240
RPT = NPAD // NS
BLK = 1024

_mesh = plsc.VectorSubcoreMesh(core_axis_name="c", subcore_axis_name="s",
                               num_cores=NC, num_subcores=NS)


def _fill(ref, n, value):
  v = jnp.full((16,), value, jnp.float32)

  def body(i, _):
    ref[pl.ds(i * 16, 16)] = v
    return 0

  lax.fori_loop(0, n // 16, body, 0)


def _deg_body(nch, row_hbm, dp_hbm, row_scr, ones_v, dbuf, deg_sh):
  c = lax.axis_index("c")
  s = lax.axis_index("s")
  w = c * NS + s
  _fill(dbuf, RPT, 0.0)
  _fill(ones_v, CHUNK, 1.0)
  pltpu.sync_copy(dbuf, deg_sh.at[pl.ds(s * RPT, RPT)])
  plsc.subcore_barrier()
  pltpu.sync_copy(row_hbm.at[w], row_scr)

  def body(j, _):
    pltpu.sync_copy(ones_v, deg_sh.at[row_scr.at[j]], add=True)
    return 0

  lax.fori_loop(0, nch, body, 0)
  plsc.subcore_barrier()
  pltpu.sync_copy(deg_sh.at[pl.ds(s * RPT, RPT)], dbuf)
  pltpu.sync_copy(dbuf, dp_hbm.at[c, pl.ds(s * RPT, RPT)])


BCH = 16


def _scat_body(nch, hs_hbm, row_hbm, col_hbm, p_hbm,
               row_scr, col_scr, b0, b1, acc_sh, s0, s1):
  c = lax.axis_index("c")
  s = lax.axis_index("s")
  w = c * NS + s
  def zbody(i, _):
    b0[i // 8, pl.ds((i % 8) * 16, 16)] = jnp.zeros((16,), jnp.float32)
    return 0

  lax.fori_loop(0, CHUNK * 8, zbody, 0)

  def ibody(k, _):
    pltpu.sync_copy(b0, acc_sh.at[pl.ds(s * RPT + k * CHUNK, CHUNK)])
    return 0

  lax.fori_loop(0, RPT // CHUNK, ibody, 0)
  plsc.subcore_barrier()

  def blk(bk, _):
    pltpu.sync_copy(row_hbm.at[w, pl.ds(bk * BCH, BCH)], row_scr)
    pltpu.sync_copy(col_hbm.at[w, pl.ds(bk * BCH, BCH)], col_scr)
    bufs = (b0, b1)
    sems = (s0, s1)
    cps = [pltpu.async_copy(hs_hbm.at[col_scr.at[0]], b0, s0), None]
    for k in range(BCH):
      if k + 1 < BCH:
        cps[(k + 1) % 2] = pltpu.async_copy(
            hs_hbm.at[col_scr.at[k + 1]], bufs[(k + 1) % 2], sems[(k + 1) % 2])
      cps[k % 2].wait()
      pltpu.sync_copy(bufs[k % 2], acc_sh.at[row_scr.at[k]], add=True)
    return 0

  lax.fori_loop(0, nch // BCH, blk, 0)
  plsc.subcore_barrier()

  def ebody(k, _):
    pltpu.sync_copy(acc_sh.at[pl.ds(s * RPT + k * CHUNK, CHUNK)], b0)
    pltpu.sync_copy(b0, p_hbm.at[c, pl.ds(s * RPT + k * CHUNK, CHUNK)])
    return 0

  lax.fori_loop(0, RPT // CHUNK, ebody, 0)


def _mm_body(x_ref, w_ref, dp_ref, hs_ref):
  i = pl.program_id(0)
  h = jnp.dot(x_ref[...], w_ref[...], preferred_element_type=jnp.float32)
  deg = dp_ref[0, pl.ds(i * BLK, BLK)] + dp_ref[1, pl.ds(i * BLK, BLK)] + 1.0
  dis = lax.rsqrt(deg)
  hs_ref[...] = dis[:, None] * h


def _fin_body(hs_ref, p_ref, dp_ref, b_ref, o_ref):
  i = pl.program_id(0)
  deg = dp_ref[0, pl.ds(i * BLK, BLK)] + dp_ref[1, pl.ds(i * BLK, BLK)] + 1.0
  dis = lax.rsqrt(deg)
  s = hs_ref[...] + p_ref[0] + p_ref[1]
  o_ref[...] = dis[:, None] * s + b_ref[...][None, :]


def kernel(x, edge_index, W, b):
  e = edge_index.shape[1]
  nch = -(-e // (NW * CHUNK))
  nch = -(-nch // BCH) * BCH
  epad = NW * nch * CHUNK
  row = edge_index[0].astype(jnp.int32)
  col = edge_index[1].astype(jnp.int32)
  pad_ix = jnp.arange(epad - e, dtype=jnp.int32)
  rowp = jnp.concatenate(
      [row, N + pad_ix % (NPAD - N)]).reshape(NW, nch, CHUNK)
  colp = jnp.concatenate(
      [col, pad_ix % N]).reshape(NW, nch, CHUNK)

  deg_call = pl.kernel(
      functools.partial(_deg_body, nch),
      out_type=jax.ShapeDtypeStruct((NC, NPAD), jnp.float32),
      mesh=_mesh,
      scratch_types=[
          pltpu.VMEM((nch, CHUNK), jnp.int32),
          pltpu.VMEM((CHUNK,), jnp.float32),
          pltpu.VMEM((RPT,), jnp.float32),
          pltpu.VMEM_SHARED((NPAD,), jnp.float32),
      ],
  )
  dp = deg_call(rowp)

  xp = jnp.concatenate([x, jnp.zeros((NPAD - N, D), x.dtype)], axis=0)
  hs = pl.pallas_call(
      _mm_body,
      out_shape=jax.ShapeDtypeStruct((NPAD, D), jnp.float32),
      grid=(NPAD // BLK,),
      in_specs=[
          pl.BlockSpec((BLK, D), lambda i: (i, 0)),
          pl.BlockSpec((D, D), lambda i: (0, 0)),
          pl.BlockSpec((NC, NPAD), lambda i: (0, 0)),
      ],
      out_specs=pl.BlockSpec((BLK, D), lambda i: (i, 0)),
  )(xp, W, dp)

  scat_call = pl.kernel(
      functools.partial(_scat_body, nch),
      out_type=jax.ShapeDtypeStruct((NC, NPAD, D), jnp.float32),
      mesh=_mesh,
      scratch_types=[
          pltpu.VMEM((BCH, CHUNK), jnp.int32),
          pltpu.VMEM((BCH, CHUNK), jnp.int32),
          pltpu.VMEM((CHUNK, D), jnp.float32),
          pltpu.VMEM((CHUNK, D), jnp.float32),
          pltpu.VMEM_SHARED((NPAD, D), jnp.float32),
          pltpu.SemaphoreType.DMA,
          pltpu.SemaphoreType.DMA,
      ],
  )
  p = scat_call(hs, rowp, colp)

  out = pl.pallas_call(
      _fin_body,
      out_shape=jax.ShapeDtypeStruct((NPAD, D), jnp.float32),
      grid=(NPAD // BLK,),
      in_specs=[
          pl.BlockSpec((BLK, D), lambda i: (i, 0)),
          pl.BlockSpec((NC, BLK, D), lambda i: (0, i, 0)),
          pl.BlockSpec((NC, NPAD), lambda i: (0, 0)),
          pl.BlockSpec((D,), lambda i: (0,)),
      ],
      out_specs=pl.BlockSpec((BLK, D), lambda i: (i, 0)),
  )(hs, p, dp, b)
  return out[:N]


# speedup vs baseline: 3.1371x; 1.0010x over previous
import functools

import jax
import jax.numpy as jnp
from jax import lax
from jax.experimental import pallas as pl
from jax.experimental.pallas import tpu as pltpu
from jax.experimental.pallas import tpu_sc as plsc

N = 10000
D = 128
NC = 2
NS = 16
NW = NC * NS
CHUNK = 128
NPAD = 10240
RPT = NPAD // NS
BLK = 1024

_mesh = plsc.VectorSubcoreMesh(core_axis_name="c", subcore_axis_name="s",
                               num_cores=NC, num_subcores=NS)


def _fill(ref, n, value):
  v = jnp.full((16,), value, jnp.float32)

  def body(i, _):
    ref[pl.ds(i * 16, 16)] = v
    return 0

  lax.fori_loop(0, n // 16, body, 0)


def _deg_body(nch, row_hbm, dp_hbm, row_scr, ones_v, dbuf, deg_sh):
  c = lax.axis_index("c")
  s = lax.axis_index("s")
  w = c * NS + s
  _fill(dbuf, RPT, 0.0)
  _fill(ones_v, CHUNK, 1.0)
  pltpu.sync_copy(dbuf, deg_sh.at[pl.ds(s * RPT, RPT)])
  plsc.subcore_barrier()
  pltpu.sync_copy(row_hbm.at[w], row_scr)

  def body(j, _):
    pltpu.sync_copy(ones_v, deg_sh.at[row_scr.at[j]], add=True)
    return 0

  lax.fori_loop(0, nch, body, 0)
  plsc.subcore_barrier()
  pltpu.sync_copy(deg_sh.at[pl.ds(s * RPT, RPT)], dbuf)
  pltpu.sync_copy(dbuf, dp_hbm.at[c, pl.ds(s * RPT, RPT)])


BCH = 16


def _scat_body(nch, hs_hbm, row_hbm, col_hbm, p_hbm,
               row_scr, col_scr, b0, b1, acc_sh, s0, s1, t0, t1):
  c = lax.axis_index("c")
  s = lax.axis_index("s")
  w = c * NS + s
  def zbody(i, _):
    b0[i // 8, pl.ds((i % 8) * 16, 16)] = jnp.zeros((16,), jnp.float32)
    return 0

  lax.fori_loop(0, CHUNK * 8, zbody, 0)

  def ibody(k, _):
    pltpu.sync_copy(b0, acc_sh.at[pl.ds(s * RPT + k * CHUNK, CHUNK)])
    return 0

  lax.fori_loop(0, RPT // CHUNK, ibody, 0)
  plsc.subcore_barrier()

  def blk(bk, _):
    pltpu.sync_copy(row_hbm.at[w, pl.ds(bk * BCH, BCH)], row_scr)
    pltpu.sync_copy(col_hbm.at[w, pl.ds(bk * BCH, BCH)], col_scr)
    bufs = (b0, b1)
    gsems = (s0, s1)
    ssems = (t0, t1)
    gcp = [pltpu.async_copy(hs_hbm.at[col_scr.at[0]], b0, s0), None]
    scp = [None, None]
    for k in range(BCH):
      if k + 1 < BCH:
        if scp[(k + 1) % 2] is not None:
          scp[(k + 1) % 2].wait()
          scp[(k + 1) % 2] = None
        gcp[(k + 1) % 2] = pltpu.async_copy(
            hs_hbm.at[col_scr.at[k + 1]], bufs[(k + 1) % 2], gsems[(k + 1) % 2])
      gcp[k % 2].wait()
      scp[k % 2] = pltpu.async_copy(
          bufs[k % 2], acc_sh.at[row_scr.at[k]], ssems[k % 2], add=True)
    for q in scp:
      if q is not None:
        q.wait()
    return 0

  lax.fori_loop(0, nch // BCH, blk, 0)
  plsc.subcore_barrier()

  def ebody(k, _):
    pltpu.sync_copy(acc_sh.at[pl.ds(s * RPT + k * CHUNK, CHUNK)], b0)
    pltpu.sync_copy(b0, p_hbm.at[c, pl.ds(s * RPT + k * CHUNK, CHUNK)])
    return 0

  lax.fori_loop(0, RPT // CHUNK, ebody, 0)


def _mm_body(x_ref, w_ref, h_ref):
  h_ref[...] = jnp.dot(x_ref[...], w_ref[...], preferred_element_type=jnp.float32)


def _scale_body(h_ref, dp_ref, hs_ref):
  i = pl.program_id(0)
  deg = dp_ref[0, pl.ds(i * BLK, BLK)] + dp_ref[1, pl.ds(i * BLK, BLK)] + 1.0
  dis = lax.rsqrt(deg)
  hs_ref[...] = dis[:, None] * h_ref[...]


def _fin_body(hs_ref, p_ref, dp_ref, b_ref, o_ref):
  i = pl.program_id(0)
  deg = dp_ref[0, pl.ds(i * BLK, BLK)] + dp_ref[1, pl.ds(i * BLK, BLK)] + 1.0
  dis = lax.rsqrt(deg)
  s = hs_ref[...] + p_ref[0] + p_ref[1]
  o_ref[...] = dis[:, None] * s + b_ref[...][None, :]


def kernel(x, edge_index, W, b):
  e = edge_index.shape[1]
  nch = -(-e // (NW * CHUNK))
  nch = -(-nch // BCH) * BCH
  epad = NW * nch * CHUNK
  row = edge_index[0].astype(jnp.int32)
  col = edge_index[1].astype(jnp.int32)
  pad_ix = jnp.arange(epad - e, dtype=jnp.int32)
  rowp = jnp.concatenate(
      [row, N + pad_ix % (NPAD - N)]).reshape(NW, nch, CHUNK)
  colp = jnp.concatenate(
      [col, pad_ix % N]).reshape(NW, nch, CHUNK)

  deg_call = pl.kernel(
      functools.partial(_deg_body, nch),
      out_type=jax.ShapeDtypeStruct((NC, NPAD), jnp.float32),
      mesh=_mesh,
      scratch_types=[
          pltpu.VMEM((nch, CHUNK), jnp.int32),
          pltpu.VMEM((CHUNK,), jnp.float32),
          pltpu.VMEM((RPT,), jnp.float32),
          pltpu.VMEM_SHARED((NPAD,), jnp.float32),
      ],
  )
  dp = deg_call(rowp)

  xp = jnp.concatenate([x, jnp.zeros((NPAD - N, D), x.dtype)], axis=0)
  h = pl.pallas_call(
      _mm_body,
      out_shape=jax.ShapeDtypeStruct((NPAD, D), jnp.float32),
      grid=(NPAD // BLK,),
      in_specs=[
          pl.BlockSpec((BLK, D), lambda i: (i, 0)),
          pl.BlockSpec((D, D), lambda i: (0, 0)),
      ],
      out_specs=pl.BlockSpec((BLK, D), lambda i: (i, 0)),
  )(xp, W)
  hs = pl.pallas_call(
      _scale_body,
      out_shape=jax.ShapeDtypeStruct((NPAD, D), jnp.float32),
      grid=(NPAD // BLK,),
      in_specs=[
          pl.BlockSpec((BLK, D), lambda i: (i, 0)),
          pl.BlockSpec((NC, NPAD), lambda i: (0, 0)),
      ],
      out_specs=pl.BlockSpec((BLK, D), lambda i: (i, 0)),
  )(h, dp)

  scat_call = pl.kernel(
      functools.partial(_scat_body, nch),
      out_type=jax.ShapeDtypeStruct((NC, NPAD, D), jnp.float32),
      mesh=_mesh,
      scratch_types=[
          pltpu.VMEM((BCH, CHUNK), jnp.int32),
          pltpu.VMEM((BCH, CHUNK), jnp.int32),
          pltpu.VMEM((CHUNK, D), jnp.float32),
          pltpu.VMEM((CHUNK, D), jnp.float32),
          pltpu.VMEM_SHARED((NPAD, D), jnp.float32),
          pltpu.SemaphoreType.DMA,
          pltpu.SemaphoreType.DMA,
          pltpu.SemaphoreType.DMA,
          pltpu.SemaphoreType.DMA,
      ],
  )
  p = scat_call(hs, rowp, colp)

  out = pl.pallas_call(
      _fin_body,
      out_shape=jax.ShapeDtypeStruct((NPAD, D), jnp.float32),
      grid=(NPAD // BLK,),
      in_specs=[
          pl.BlockSpec((BLK, D), lambda i: (i, 0)),
          pl.BlockSpec((NC, BLK, D), lambda i: (0, i, 0)),
          pl.BlockSpec((NC, NPAD), lambda i: (0, 0)),
          pl.BlockSpec((D,), lambda i: (0,)),
      ],
      out_specs=pl.BlockSpec((BLK, D), lambda i: (i, 0)),
  )(hs, p, dp, b)
  return out[:N]
